# baseline (speedup 1.0000x reference)
import functools

import jax
import jax.numpy as jnp
import numpy as np
from jax.experimental import pallas as pl
from jax.experimental.pallas import tpu as pltpu

VMEM_LIMIT = 64 * 1024 * 1024


def _support_kernel(x_ref, w_ref, o_ref):
    o_ref[...] = jnp.dot(
        x_ref[...].astype(jnp.bfloat16), w_ref[...],
        preferred_element_type=jnp.float32).astype(jnp.bfloat16)


def _compute_support(x, w_bf16, *, tm):
    n, nfeat = x.shape
    nhid = w_bf16.shape[1]
    return pl.pallas_call(
        _support_kernel,
        out_shape=jax.ShapeDtypeStruct((n, nhid), jnp.bfloat16),
        grid=(n // tm,),
        in_specs=[
            pl.BlockSpec((tm, nfeat), lambda i: (i, 0)),
            pl.BlockSpec((nfeat, nhid), lambda i: (0, 0)),
        ],
        out_specs=pl.BlockSpec((tm, nhid), lambda i: (i, 0)),
        compiler_params=pltpu.CompilerParams(
            dimension_semantics=("arbitrary",),
            vmem_limit_bytes=VMEM_LIMIT),
    )(x, w_bf16)


def _gcn_kernel(ia_ref, ja_ref, ib_ref, jb_ref, vb_ref, e1_ref, e2_ref,
                s1_ref, blka_ref, blkb_ref, w2_ref,
                b1_ref, b2_ref, out_ref,
                tri_res, s2_buf, s2t_buf,
                h1d, h1t, outd, outt, *, nb, bs, n_steps):
    t = pl.program_id(0)
    dn_ta = (((0,), (0,)), ((), ()))

    @pl.when(t == 0)
    def _():
        h1d[...] = jnp.zeros_like(h1d)
        h1t[...] = jnp.zeros_like(h1t)
        outd[...] = jnp.zeros_like(outd)
        outt[...] = jnp.zeros_like(outt)

    @pl.when(t > 0)
    def _():
        def process(blk_ref, i, j, pos):
            a = blk_ref[...].astype(jnp.bfloat16)
            tri_res[pos] = a
            h1d[i] += jnp.dot(a, s1_ref[pl.ds(j * bs, bs), :],
                              preferred_element_type=jnp.float32)
            @pl.when(i != j)
            def _():
                h1t[j] += jax.lax.dot_general(
                    s1_ref[pl.ds(i * bs, bs), :], a, dn_ta,
                    preferred_element_type=jnp.float32)

        pos_a = 2 * (t - 1)
        process(blka_ref, ia_ref[t], ja_ref[t], pos_a)
        @pl.when(vb_ref[t] == 1)
        def _():
            process(blkb_ref, ib_ref[t], jb_ref[t], pos_a + 1)

        def col_work(r):
            h1_r = h1d[r] + h1t[r].T + b1_ref[...]
            h1_r = jnp.maximum(h1_r, 0.0).astype(jnp.bfloat16)
            s2_r = jnp.dot(h1_r, w2_ref[...],
                           preferred_element_type=jnp.float32
                           ).astype(jnp.bfloat16)
            s2_buf[r] = s2_r
            s2t_buf[r] = s2_r.T
            for i2 in range(nb):
                base = i2 * nb - (i2 * (i2 - 1)) // 2 - i2
                @pl.when(i2 <= r)
                def _():
                    blk = tri_res[base + r]
                    outd[i2] += jnp.dot(blk, s2_buf[r],
                                        preferred_element_type=jnp.float32)
                    @pl.when(i2 != r)
                    def _():
                        outt[r] += jnp.dot(s2t_buf[i2], blk,
                                           preferred_element_type=jnp.float32)

        @pl.when(e1_ref[t] >= 0)
        def _():
            col_work(e1_ref[t])
        @pl.when(e2_ref[t] >= 0)
        def _():
            col_work(e2_ref[t])

        @pl.when(t == n_steps - 1)
        def _():
            b2 = b2_ref[...]
            for m in range(nb):
                out_ref[m * bs:(m + 1) * bs, :] = outd[m] + outt[m].T + b2


def kernel(feature, adj, w1, b1, w2, b2):
    n, nfeat = feature.shape
    nhid1 = w1.shape[1]
    nhid2 = w2.shape[1]

    w1_bf = w1.astype(jnp.bfloat16)
    w2_bf = w2.astype(jnp.bfloat16)
    b1_2d = b1.reshape(1, nhid1).astype(jnp.float32)
    b2_2d = b2.reshape(1, nhid2).astype(jnp.float32)

    s1 = _compute_support(feature, w1_bf, tm=n // 4)

    bs = 1024
    nb = n // bs
    tri = [(i, j) for i in range(nb) for j in range(i, nb)]
    n_pos = len(tri)
    n_pairs = (n_pos + 1) // 2
    n_steps = n_pairs + 1

    ia, ja, ib, jb, vb = [], [], [], [], []
    for p in range(0, 2 * n_pairs, 2):
        ia.append(tri[p][0]); ja.append(tri[p][1])
        if p + 1 < n_pos:
            ib.append(tri[p + 1][0]); jb.append(tri[p + 1][1]); vb.append(1)
        else:
            ib.append(tri[p][0]); jb.append(tri[p][1]); vb.append(0)
    ia = [ia[0]] + ia; ja = [ja[0]] + ja
    ib = [ib[0]] + ib; jb = [jb[0]] + jb; vb = [0] + vb

    e1 = [-1] * n_steps
    e2 = [-1] * n_steps
    for r in range(nb):
        p_end = sum(nb - k for k in range(r + 1)) - 1
        s = 1 + p_end // 2
        if e1[s] < 0:
            e1[s] = r
        else:
            e2[s] = r

    arrs = [np.asarray(a, dtype=np.int32)
            for a in (ia, ja, ib, jb, vb, e1, e2)]

    body = functools.partial(_gcn_kernel, nb=nb, bs=bs, n_steps=n_steps)
    const = lambda t, iA, jA, iB, jB, vB, E1, E2: (0, 0)
    out = pl.pallas_call(
        body,
        out_shape=jax.ShapeDtypeStruct((n, nhid2), jnp.float32),
        grid_spec=pltpu.PrefetchScalarGridSpec(
            num_scalar_prefetch=7,
            grid=(n_steps,),
            in_specs=[
                pl.BlockSpec((n, nhid1), const),
                pl.BlockSpec(
                    (bs, bs),
                    lambda t, iA, jA, iB, jB, vB, E1, E2: (iA[t], jA[t])),
                pl.BlockSpec(
                    (bs, bs),
                    lambda t, iA, jA, iB, jB, vB, E1, E2: (iB[t], jB[t])),
                pl.BlockSpec((nhid1, nhid2), const),
                pl.BlockSpec((1, nhid1), const),
                pl.BlockSpec((1, nhid2), const),
            ],
            out_specs=pl.BlockSpec((n, nhid2), const),
            scratch_shapes=[
                pltpu.VMEM((n_pos, bs, bs), jnp.bfloat16),
                pltpu.VMEM((nb, bs, nhid2), jnp.bfloat16),
                pltpu.VMEM((nb, nhid2, bs), jnp.bfloat16),
                pltpu.VMEM((nb, bs, nhid1), jnp.float32),
                pltpu.VMEM((nb, nhid1, bs), jnp.float32),
                pltpu.VMEM((nb, bs, nhid2), jnp.float32),
                pltpu.VMEM((nb, nhid2, bs), jnp.float32),
            ],
        ),
        compiler_params=pltpu.CompilerParams(
            dimension_semantics=("arbitrary",),
            vmem_limit_bytes=VMEM_LIMIT),
    )(*arrs, s1, adj, adj, w2_bf, b1_2d, b2_2d)
    return out


# speedup vs baseline: 1.1376x; 1.1376x over previous
import functools

import jax
import jax.numpy as jnp
from jax.experimental import pallas as pl
from jax.experimental.pallas import tpu as pltpu

VMEM_LIMIT = 64 * 1024 * 1024


def _gcn_kernel(x_ref, adj_lo_ref, adj_hi_ref, w1_ref, w2_ref, b1_ref,
                b2_ref, out_ref, s1_ref, acc_ref, *, n_steps):
    t = pl.program_id(0)

    @pl.when(t == 0)
    def _():
        s1_ref[...] = jnp.dot(
            x_ref[...].astype(jnp.bfloat16), w1_ref[...],
            preferred_element_type=jnp.float32).astype(jnp.bfloat16)

    @pl.when(t > 0)
    def _():
        def partial(adj_slab_ref):
            a = adj_slab_ref[...].astype(jnp.bfloat16)
            h1 = jnp.dot(a, s1_ref[...], preferred_element_type=jnp.float32)
            h1 = jnp.maximum(h1 + b1_ref[...], 0.0).astype(jnp.bfloat16)
            s2_t = jnp.dot(
                h1, w2_ref[...],
                preferred_element_type=jnp.float32).astype(jnp.bfloat16)
            dn = (((0,), (0,)), ((), ()))
            return jax.lax.dot_general(
                s2_t, a, dn, preferred_element_type=jnp.float32)

        pm = partial(adj_lo_ref) + partial(adj_hi_ref)
        @pl.when(t == 1)
        def _():
            acc_ref[...] = pm
        @pl.when(t > 1)
        def _():
            acc_ref[...] += pm

        @pl.when(t == n_steps - 1)
        def _():
            out_ref[...] = acc_ref[...].T + b2_ref[...]


def kernel(feature, adj, w1, b1, w2, b2):
    n, nfeat = feature.shape
    nhid1 = w1.shape[1]
    nhid2 = w2.shape[1]

    w1_bf = w1.astype(jnp.bfloat16)
    w2_bf = w2.astype(jnp.bfloat16)
    b1_2d = b1.reshape(1, nhid1).astype(jnp.float32)
    b2_2d = b2.reshape(1, nhid2).astype(jnp.float32)

    slab = 512
    half_slabs = n // (2 * slab)
    n_steps = half_slabs + 1

    body = functools.partial(_gcn_kernel, n_steps=n_steps)
    lo = lambda t: (jnp.maximum(t - 1, 0), 0)
    hi = lambda t: (half_slabs + jnp.maximum(t - 1, 0), 0)
    out = pl.pallas_call(
        body,
        out_shape=jax.ShapeDtypeStruct((n, nhid2), jnp.float32),
        grid=(n_steps,),
        in_specs=[
            pl.BlockSpec((n, nfeat), lambda t: (0, 0)),
            pl.BlockSpec((slab, n), lo),
            pl.BlockSpec((slab, n), hi),
            pl.BlockSpec((nfeat, nhid1), lambda t: (0, 0)),
            pl.BlockSpec((nhid1, nhid2), lambda t: (0, 0)),
            pl.BlockSpec((1, nhid1), lambda t: (0, 0)),
            pl.BlockSpec((1, nhid2), lambda t: (0, 0)),
        ],
        out_specs=pl.BlockSpec((n, nhid2), lambda t: (0, 0)),
        scratch_shapes=[
            pltpu.VMEM((n, nhid1), jnp.bfloat16),
            pltpu.VMEM((nhid2, n), jnp.float32),
        ],
        compiler_params=pltpu.CompilerParams(
            dimension_semantics=("arbitrary",),
            vmem_limit_bytes=VMEM_LIMIT),
    )(feature, adj, adj, w1_bf, w2_bf, b1_2d, b2_2d)
    return out
